# Initial kernel scaffold; baseline (speedup 1.0000x reference)
import jax
import jax.numpy as jnp
from jax.experimental import pallas as pl


def kernel(X, chain_idx, mask, E_idx, D_neighbors, pos_W, pos_b, edge_W, ln_gamma, ln_beta):
    raise NotImplementedError("write your pallas kernel here")


# speedup vs baseline: 1.0123x; 1.0123x over previous
import functools

import numpy as np
import jax
import jax.numpy as jnp
from jax import lax
from jax.experimental import pallas as pl
from jax.experimental.pallas import tpu as pltpu
from jax.experimental.pallas import tpu_sc as plsc

NUM_RBF = 16
D_MIN, D_MAX = 2.0, 22.0
SIGMA = (D_MAX - D_MIN) / NUM_RBF

_PAIRS = [(0, 0), (2, 2), (3, 3), (4, 4), (1, 0), (1, 2), (1, 3), (1, 4),
          (0, 2), (0, 3), (0, 4), (4, 2), (4, 3), (3, 2), (0, 1), (2, 1),
          (3, 1), (4, 1), (2, 0), (3, 0), (4, 0), (2, 4), (3, 4), (2, 3)]

_BLOCK_OF_Q = np.zeros(25, np.int32)
for _p, (_a, _b) in enumerate(_PAIRS):
    _BLOCK_OF_Q[5 * _a + _b] = _p + 1
_BLOCK_OF_Q[6] = 0

_UA = np.zeros((16, 75), np.float32)
_UB = np.zeros((16, 75), np.float32)
_V3 = np.zeros((75, 25), np.float32)
_USA = np.zeros((16, 25), np.float32)
_UNB = np.zeros((16, 25), np.float32)
for _q in range(25):
    _a, _b = _q // 5, _q % 5
    for _c in range(3):
        _UA[3 * _a + _c, 3 * _q + _c] = 1.0
        _UB[3 * _b + _c, 3 * _q + _c] = 1.0
        _V3[3 * _q + _c, _q] = 1.0
        _USA[3 * _a + _c, _q] = 1.0
        _UNB[3 * _b + _c, _q] = 1.0

_MU = np.linspace(D_MIN, D_MAX, NUM_RBF).astype(np.float32)
_SEXP2 = np.zeros((25, 400), np.float32)
_MUS = np.zeros((1, 400), np.float32)
for _q in range(25):
    _blk = int(_BLOCK_OF_Q[_q])
    for _r in range(NUM_RBF):
        _SEXP2[_q, 16 * _blk + _r] = 1.0 / SIGMA
for _blk in range(25):
    _MUS[0, 16 * _blk:16 * _blk + 16] = _MU / SIGMA


def _dense_body(self_ref, gath_ref, dn_ref, ua_ref, ub_ref, v3_ref, usa_ref,
                unb_ref, sexp_ref, mus_ref, w2_ref, pt_ref, gam_ref, bet_ref,
                out_ref):
    f32 = jnp.float32
    S = self_ref[...]
    N = gath_ref[...]
    dot = functools.partial(jnp.dot, preferred_element_type=f32,
                            precision=lax.Precision.HIGHEST)

    sa = dot(S, ua_ref[...])
    nb = dot(N, ub_ref[...])
    g = dot(sa * nb, v3_ref[...])
    ssq = dot(S * S, usa_ref[...])
    nsq = dot(N * N, unb_ref[...])
    d2 = ssq + nsq - 2.0 * g
    dist = jnp.sqrt(jnp.maximum(d2, 0.0) + 1e-06)

    T = dist.shape[0]
    lane25 = lax.broadcasted_iota(jnp.int32, (T, 25), 1)
    dn = jnp.broadcast_to(dn_ref[...], (T, 25))
    dfull = jnp.where(lane25 == 6, dn, dist)

    z = dot(dfull, sexp_ref[...]) - mus_ref[...]
    rbf = jnp.exp(-(z * z))
    e = dot(rbf, w2_ref[...])

    ci = S[:, 15:16].astype(jnp.int32)
    cj = N[:, 15:16].astype(jnp.int32)
    res_i, chain_i = ci >> 2, ci & 3
    res_j, chain_j = cj >> 2, cj & 3
    off = res_i - res_j
    d = jnp.where(chain_i == chain_j,
                  jnp.clip(off + 32, 0, 64), 65)
    lane72 = lax.broadcasted_iota(jnp.int32, (T, 72), 1)
    onehot = (lane72 == d).astype(f32)
    e = e + dot(onehot, pt_ref[...])

    mu = jnp.mean(e, axis=-1, keepdims=True)
    c = e - mu
    var = jnp.mean(c * c, axis=-1, keepdims=True)
    out_ref[...] = c * lax.rsqrt(var + 1e-05) * gam_ref[...] + bet_ref[...]


_TILE = 512


def _dense_call(self_exp, gath, dn, w2p, pt, gamma, beta):
    M = self_exp.shape[0]
    grid = (M // _TILE,)
    consts = (jnp.asarray(_UA), jnp.asarray(_UB), jnp.asarray(_V3),
              jnp.asarray(_USA), jnp.asarray(_UNB), jnp.asarray(_SEXP2),
              jnp.asarray(_MUS))
    edge = lambda i: (i, 0)
    full = lambda i: (0, 0)
    in_specs = [
        pl.BlockSpec((_TILE, 16), edge),
        pl.BlockSpec((_TILE, 16), edge),
        pl.BlockSpec((_TILE, 1), edge),
        pl.BlockSpec((16, 75), full),
        pl.BlockSpec((16, 75), full),
        pl.BlockSpec((75, 25), full),
        pl.BlockSpec((16, 25), full),
        pl.BlockSpec((16, 25), full),
        pl.BlockSpec((25, 400), full),
        pl.BlockSpec((1, 400), full),
        pl.BlockSpec((400, 128), full),
        pl.BlockSpec((72, 128), full),
        pl.BlockSpec((1, 128), full),
        pl.BlockSpec((1, 128), full),
    ]
    return pl.pallas_call(
        _dense_body,
        grid=grid,
        in_specs=in_specs,
        out_specs=pl.BlockSpec((_TILE, 128), edge),
        out_shape=jax.ShapeDtypeStruct((M, 128), jnp.float32),
    )(self_exp, gath, dn, *consts, w2p, pt, gamma, beta)


_NC, _NS = 2, 16
_NW = _NC * _NS
_CHUNK = 128


def _sc_gather(table, idx):
    M = idx.shape[0]
    per_w = M // _NW
    nch = per_w // _CHUNK
    idx3 = idx.reshape(_NW, nch, _CHUNK)
    mesh = plsc.VectorSubcoreMesh(core_axis_name="c", subcore_axis_name="s",
                                  num_cores=_NC, num_subcores=_NS)

    @functools.partial(
        pl.kernel,
        out_type=jax.ShapeDtypeStruct((M, 16), jnp.float32),
        mesh=mesh,
        compiler_params=pltpu.CompilerParams(use_tc_tiling_on_sc=False),
        scratch_types=[
            pltpu.VMEM((nch, _CHUNK), jnp.int32),
            pltpu.VMEM((per_w, 16), jnp.float32),
            pltpu.SemaphoreType.DMA,
        ],
    )
    def gather_k(table_hbm, idx_hbm, out_hbm, idx_v, rows_v, sem):
        wid = lax.axis_index("s") * _NC + lax.axis_index("c")
        pltpu.sync_copy(idx_hbm.at[wid], idx_v)
        for g in range(0, nch, 10):
            handles = [
                pltpu.async_copy(table_hbm.at[idx_v.at[c]],
                                 rows_v.at[pl.ds(c * _CHUNK, _CHUNK)], sem)
                for c in range(g, min(g + 10, nch))
            ]
            for h in handles:
                h.wait()
        pltpu.sync_copy(rows_v, out_hbm.at[pl.ds(wid * per_w, per_w)])

    return gather_k(table, idx3)


def _build_table(X, chain_idx):
    Nat = X[:, :, 0, :]
    Ca = X[:, :, 1, :]
    C = X[:, :, 2, :]
    O = X[:, :, 3, :]
    b = Ca - Nat
    c = C - Ca
    a = jnp.cross(b, c)
    Cb = -0.58273431 * a + 0.56802827 * b - 0.54067466 * c + Ca

    Bsz, L = chain_idx.shape
    change = jnp.concatenate(
        [jnp.zeros((Bsz, 1), dtype=jnp.int32),
         (chain_idx[:, 1:] != chain_idx[:, :-1]).astype(jnp.int32)], axis=1)
    rank = jnp.cumsum(change, axis=1)
    res = 100 * rank + jnp.arange(L, dtype=jnp.int32)[None, :]
    res = jnp.where(rank == rank[:, -1:], -100, res)
    combined = (4 * res + chain_idx.astype(jnp.int32)).astype(jnp.float32)

    return jnp.concatenate([Nat, Ca, C, O, Cb, combined[..., None]], axis=-1)


def kernel(X, chain_idx, mask, E_idx, D_neighbors,
           pos_W, pos_b, edge_W, ln_gamma, ln_beta):
    B, L, K = E_idx.shape
    M = B * L * K

    table = _build_table(X, chain_idx)
    table_flat = table.reshape(B * L, 16)
    idx_flat = (E_idx.astype(jnp.int32)
                + (jnp.arange(B, dtype=jnp.int32) * L)[:, None, None])
    gath = _sc_gather(table_flat, idx_flat.reshape(M))

    self_exp = jnp.broadcast_to(table[:, :, None, :], (B, L, K, 16))
    self_exp = self_exp.reshape(M, 16)
    dn = D_neighbors.reshape(M, 1)

    W1 = edge_W[:, :16]
    pt = (pos_W.T + pos_b[None, :]) @ W1.T
    pt = jnp.concatenate(
        [pt, jnp.zeros((6, 128), jnp.float32)], axis=0)
    w2p = edge_W[:, 16:].T

    out = _dense_call(self_exp, gath, dn, w2p, pt,
                      ln_gamma.reshape(1, 128), ln_beta.reshape(1, 128))
    return out.reshape(B, L, K, 128)


# speedup vs baseline: 1.4913x; 1.4732x over previous
import functools

import numpy as np
import jax
import jax.numpy as jnp
from jax import lax
from jax.experimental import pallas as pl
from jax.experimental.pallas import tpu as pltpu
from jax.experimental.pallas import tpu_sc as plsc

NUM_RBF = 16
D_MIN, D_MAX = 2.0, 22.0
SIGMA = (D_MAX - D_MIN) / NUM_RBF

_PAIRS = [(0, 0), (2, 2), (3, 3), (4, 4), (1, 0), (1, 2), (1, 3), (1, 4),
          (0, 2), (0, 3), (0, 4), (4, 2), (4, 3), (3, 2), (0, 1), (2, 1),
          (3, 1), (4, 1), (2, 0), (3, 0), (4, 0), (2, 4), (3, 4), (2, 3)]

_BLOCK_OF_Q = np.zeros(25, np.int32)
for _p, (_a, _b) in enumerate(_PAIRS):
    _BLOCK_OF_Q[5 * _a + _b] = _p + 1
_BLOCK_OF_Q[6] = 0

_UA = np.zeros((16, 75), np.float32)
_UB = np.zeros((16, 75), np.float32)
_V3 = np.zeros((75, 25), np.float32)
_USA = np.zeros((16, 25), np.float32)
_UNB = np.zeros((16, 25), np.float32)
for _q in range(25):
    _a, _b = _q // 5, _q % 5
    for _c in range(3):
        _UA[3 * _a + _c, 3 * _q + _c] = 1.0
        _UB[3 * _b + _c, 3 * _q + _c] = 1.0
        _V3[3 * _q + _c, _q] = 1.0
        _USA[3 * _a + _c, _q] = 1.0
        _UNB[3 * _b + _c, _q] = 1.0

_MU = np.linspace(D_MIN, D_MAX, NUM_RBF).astype(np.float32)
_SEXP2 = np.zeros((25, 400), np.float32)
_MUS = np.zeros((1, 400), np.float32)
for _q in range(25):
    _blk = int(_BLOCK_OF_Q[_q])
    for _r in range(NUM_RBF):
        _SEXP2[_q, 16 * _blk + _r] = 1.0
for _blk in range(25):
    _MUS[0, 16 * _blk:16 * _blk + 16] = _MU
_INV_SIGMA = 1.0 / SIGMA


def _dense_body(self_ref, gath_ref, dn_ref, ua_ref, ub_ref, v3_ref, usa_ref,
                unb_ref, sexp_ref, mus_ref, w2_ref, pt_ref, gam_ref, bet_ref,
                out_ref):
    f32 = jnp.float32
    S = self_ref[...]
    N = gath_ref[...]
    dot = functools.partial(jnp.dot, preferred_element_type=f32,
                            precision=lax.Precision.HIGHEST)
    dotb = functools.partial(jnp.dot, preferred_element_type=f32,
                             precision=lax.Precision.DEFAULT)

    sa = dot(S, ua_ref[...])
    nb = dot(N, ub_ref[...])
    g = dot(sa * nb, v3_ref[...])
    ssq = dot(S * S, usa_ref[...])
    nsq = dot(N * N, unb_ref[...])
    d2 = ssq + nsq - 2.0 * g
    dist = jnp.sqrt(jnp.maximum(d2, 0.0) + 1e-06)

    T = dist.shape[0]
    lane25 = lax.broadcasted_iota(jnp.int32, (T, 25), 1)
    dn = jnp.broadcast_to(dn_ref[...], (T, 25))
    dfull = jnp.where(lane25 == 6, dn, dist)

    dhi = dfull.astype(jnp.bfloat16)
    dlo = (dfull - dhi.astype(f32)).astype(jnp.bfloat16)
    t = dotb(dhi, sexp_ref[...]) + dotb(dlo, sexp_ref[...])
    z = (t - mus_ref[...]) * _INV_SIGMA
    rbf = jnp.exp(-(z * z))
    e = dotb(rbf.astype(jnp.bfloat16), w2_ref[...])

    ci = S[:, 15:16].astype(jnp.int32)
    cj = N[:, 15:16].astype(jnp.int32)
    res_i, chain_i = ci >> 2, ci & 3
    res_j, chain_j = cj >> 2, cj & 3
    off = res_i - res_j
    d = jnp.where(chain_i == chain_j,
                  jnp.clip(off + 32, 0, 64), 65)
    lane72 = lax.broadcasted_iota(jnp.int32, (T, 72), 1)
    onehot = (lane72 == d).astype(jnp.bfloat16)
    e = e + dotb(onehot, pt_ref[...])

    mu = jnp.mean(e, axis=-1, keepdims=True)
    c = e - mu
    var = jnp.mean(c * c, axis=-1, keepdims=True)
    out_ref[...] = c * lax.rsqrt(var + 1e-05) * gam_ref[...] + bet_ref[...]


_TILE = 512


def _dense_call(self_exp, gath, dn, w2p, pt, gamma, beta):
    M = self_exp.shape[0]
    grid = (M // _TILE,)
    consts = (jnp.asarray(_UA), jnp.asarray(_UB), jnp.asarray(_V3),
              jnp.asarray(_USA), jnp.asarray(_UNB),
              jnp.asarray(_SEXP2, dtype=jnp.bfloat16), jnp.asarray(_MUS))
    edge = lambda i: (i, 0)
    full = lambda i: (0, 0)
    in_specs = [
        pl.BlockSpec((_TILE, 16), edge),
        pl.BlockSpec((_TILE, 16), edge),
        pl.BlockSpec((_TILE, 1), edge),
        pl.BlockSpec((16, 75), full),
        pl.BlockSpec((16, 75), full),
        pl.BlockSpec((75, 25), full),
        pl.BlockSpec((16, 25), full),
        pl.BlockSpec((16, 25), full),
        pl.BlockSpec((25, 400), full),
        pl.BlockSpec((1, 400), full),
        pl.BlockSpec((400, 128), full),
        pl.BlockSpec((72, 128), full),
        pl.BlockSpec((1, 128), full),
        pl.BlockSpec((1, 128), full),
    ]
    return pl.pallas_call(
        _dense_body,
        grid=grid,
        in_specs=in_specs,
        out_specs=pl.BlockSpec((_TILE, 128), edge),
        out_shape=jax.ShapeDtypeStruct((M, 128), jnp.float32),
    )(self_exp, gath, dn, *consts, w2p, pt, gamma, beta)


_NC, _NS = 2, 16
_NW = _NC * _NS
_CHUNK = 128


def _sc_gather(table, idx):
    M = idx.shape[0]
    per_w = M // _NW
    nch = per_w // _CHUNK
    idx3 = idx.reshape(_NW, nch, _CHUNK)
    mesh = plsc.VectorSubcoreMesh(core_axis_name="c", subcore_axis_name="s",
                                  num_cores=_NC, num_subcores=_NS)

    @functools.partial(
        pl.kernel,
        out_type=jax.ShapeDtypeStruct((M, 16), jnp.float32),
        mesh=mesh,
        compiler_params=pltpu.CompilerParams(use_tc_tiling_on_sc=False),
        scratch_types=[
            pltpu.VMEM((nch, _CHUNK), jnp.int32),
            pltpu.VMEM((per_w, 16), jnp.float32),
            pltpu.SemaphoreType.DMA,
        ],
    )
    def gather_k(table_hbm, idx_hbm, out_hbm, idx_v, rows_v, sem):
        wid = lax.axis_index("s") * _NC + lax.axis_index("c")
        pltpu.sync_copy(idx_hbm.at[wid], idx_v)
        for g in range(0, nch, 10):
            handles = [
                pltpu.async_copy(table_hbm.at[idx_v.at[c]],
                                 rows_v.at[pl.ds(c * _CHUNK, _CHUNK)], sem)
                for c in range(g, min(g + 10, nch))
            ]
            for h in handles:
                h.wait()
        pltpu.sync_copy(rows_v, out_hbm.at[pl.ds(wid * per_w, per_w)])

    return gather_k(table, idx3)


def _build_table(X, chain_idx):
    Nat = X[:, :, 0, :]
    Ca = X[:, :, 1, :]
    C = X[:, :, 2, :]
    O = X[:, :, 3, :]
    b = Ca - Nat
    c = C - Ca
    a = jnp.cross(b, c)
    Cb = -0.58273431 * a + 0.56802827 * b - 0.54067466 * c + Ca

    Bsz, L = chain_idx.shape
    change = jnp.concatenate(
        [jnp.zeros((Bsz, 1), dtype=jnp.int32),
         (chain_idx[:, 1:] != chain_idx[:, :-1]).astype(jnp.int32)], axis=1)
    rank = jnp.cumsum(change, axis=1)
    res = 100 * rank + jnp.arange(L, dtype=jnp.int32)[None, :]
    res = jnp.where(rank == rank[:, -1:], -100, res)
    combined = (4 * res + chain_idx.astype(jnp.int32)).astype(jnp.float32)

    return jnp.concatenate([Nat, Ca, C, O, Cb, combined[..., None]], axis=-1)


def kernel(X, chain_idx, mask, E_idx, D_neighbors,
           pos_W, pos_b, edge_W, ln_gamma, ln_beta):
    B, L, K = E_idx.shape
    M = B * L * K

    table = _build_table(X, chain_idx)
    table_flat = table.reshape(B * L, 16)
    idx_flat = (E_idx.astype(jnp.int32)
                + (jnp.arange(B, dtype=jnp.int32) * L)[:, None, None])
    gath = _sc_gather(table_flat, idx_flat.reshape(M))

    self_exp = jnp.broadcast_to(table[:, :, None, :], (B, L, K, 16))
    self_exp = self_exp.reshape(M, 16)
    dn = D_neighbors.reshape(M, 1)

    W1 = edge_W[:, :16]
    pt = (pos_W.T + pos_b[None, :]) @ W1.T
    pt = jnp.concatenate(
        [pt, jnp.zeros((6, 128), jnp.float32)], axis=0)
    pt = pt.astype(jnp.bfloat16)
    w2p = edge_W[:, 16:].T.astype(jnp.bfloat16)

    out = _dense_call(self_exp, gath, dn, w2p, pt,
                      ln_gamma.reshape(1, 128), ln_beta.reshape(1, 128))
    return out.reshape(B, L, K, 128)


# speedup vs baseline: 1.5869x; 1.0641x over previous
import functools

import numpy as np
import jax
import jax.numpy as jnp
from jax import lax
from jax.experimental import pallas as pl
from jax.experimental.pallas import tpu as pltpu
from jax.experimental.pallas import tpu_sc as plsc

NUM_RBF = 16
D_MIN, D_MAX = 2.0, 22.0
SIGMA = (D_MAX - D_MIN) / NUM_RBF

_PAIRS = [(0, 0), (2, 2), (3, 3), (4, 4), (1, 0), (1, 2), (1, 3), (1, 4),
          (0, 2), (0, 3), (0, 4), (4, 2), (4, 3), (3, 2), (0, 1), (2, 1),
          (3, 1), (4, 1), (2, 0), (3, 0), (4, 0), (2, 4), (3, 4), (2, 3)]

_BLOCK_OF_Q = np.zeros(25, np.int32)
for _p, (_a, _b) in enumerate(_PAIRS):
    _BLOCK_OF_Q[5 * _a + _b] = _p + 1
_BLOCK_OF_Q[6] = 0

_UA = np.zeros((16, 75), np.float32)
_UB = np.zeros((16, 75), np.float32)
_V3 = np.zeros((75, 25), np.float32)
_USA = np.zeros((16, 25), np.float32)
_UNB = np.zeros((16, 25), np.float32)
for _q in range(25):
    _a, _b = _q // 5, _q % 5
    for _c in range(3):
        _UA[3 * _a + _c, 3 * _q + _c] = 1.0
        _UB[3 * _b + _c, 3 * _q + _c] = 1.0
        _V3[3 * _q + _c, _q] = 1.0
        _USA[3 * _a + _c, _q] = 1.0
        _UNB[3 * _b + _c, _q] = 1.0

_MU = np.linspace(D_MIN, D_MAX, NUM_RBF).astype(np.float32)
_SEXP2 = np.zeros((25, 400), np.float32)
_MUS = np.zeros((1, 400), np.float32)
for _q in range(25):
    _blk = int(_BLOCK_OF_Q[_q])
    for _r in range(NUM_RBF):
        _SEXP2[_q, 16 * _blk + _r] = 1.0
for _blk in range(25):
    _MUS[0, 16 * _blk:16 * _blk + 16] = _MU
_INV_SIGMA = 1.0 / SIGMA


def _dense_body(self_ref, gath_ref, dn_ref, ua_ref, ub_ref, v3_ref, usa_ref,
                unb_ref, sexp_ref, mus_ref, w2_ref, pt_ref, gam_ref, bet_ref,
                out_ref):
    f32 = jnp.float32
    S = self_ref[...]
    N = gath_ref[...]
    dot = functools.partial(jnp.dot, preferred_element_type=f32,
                            precision=lax.Precision.HIGHEST)
    dotb = functools.partial(jnp.dot, preferred_element_type=f32,
                             precision=lax.Precision.DEFAULT)

    sa = dot(S, ua_ref[...])
    nb = dot(N, ub_ref[...])
    g = dot(sa * nb, v3_ref[...])
    ssq = dot(S * S, usa_ref[...])
    nsq = dot(N * N, unb_ref[...])
    d2 = ssq + nsq - 2.0 * g
    dist = jnp.sqrt(jnp.maximum(d2, 0.0) + 1e-06)

    T = dist.shape[0]
    lane25 = lax.broadcasted_iota(jnp.int32, (T, 25), 1)
    dn = jnp.broadcast_to(dn_ref[...], (T, 25))
    dfull = jnp.where(lane25 == 6, dn, dist)

    dhi = dfull.astype(jnp.bfloat16)
    dlo = (dfull - dhi.astype(f32)).astype(jnp.bfloat16)
    t = dotb(dhi, sexp_ref[...]) + dotb(dlo, sexp_ref[...])
    z = (t - mus_ref[...]) * _INV_SIGMA
    rbf = jnp.exp(-(z * z))
    e = dotb(rbf.astype(jnp.bfloat16), w2_ref[...])

    ci = S[:, 15:16].astype(jnp.int32)
    cj = N[:, 15:16].astype(jnp.int32)
    res_i, chain_i = ci >> 2, ci & 3
    res_j, chain_j = cj >> 2, cj & 3
    off = res_i - res_j
    d = jnp.where(chain_i == chain_j,
                  jnp.clip(off + 32, 0, 64), 65)
    lane72 = lax.broadcasted_iota(jnp.int32, (T, 72), 1)
    onehot = (lane72 == d).astype(jnp.bfloat16)
    e = e + dotb(onehot, pt_ref[...])

    mu = jnp.mean(e, axis=-1, keepdims=True)
    c = e - mu
    var = jnp.mean(c * c, axis=-1, keepdims=True)
    out_ref[...] = c * lax.rsqrt(var + 1e-05) * gam_ref[...] + bet_ref[...]


_TILE = 1024


def _dense_call(self_exp, gath, dn, w2p, pt, gamma, beta):
    M = self_exp.shape[0]
    grid = (M // _TILE,)
    consts = (jnp.asarray(_UA), jnp.asarray(_UB), jnp.asarray(_V3),
              jnp.asarray(_USA), jnp.asarray(_UNB),
              jnp.asarray(_SEXP2, dtype=jnp.bfloat16), jnp.asarray(_MUS))
    edge = lambda i: (i, 0)
    full = lambda i: (0, 0)
    in_specs = [
        pl.BlockSpec((_TILE, 16), edge),
        pl.BlockSpec((_TILE, 16), edge),
        pl.BlockSpec((_TILE, 1), edge),
        pl.BlockSpec((16, 75), full),
        pl.BlockSpec((16, 75), full),
        pl.BlockSpec((75, 25), full),
        pl.BlockSpec((16, 25), full),
        pl.BlockSpec((16, 25), full),
        pl.BlockSpec((25, 400), full),
        pl.BlockSpec((1, 400), full),
        pl.BlockSpec((400, 128), full),
        pl.BlockSpec((72, 128), full),
        pl.BlockSpec((1, 128), full),
        pl.BlockSpec((1, 128), full),
    ]
    return pl.pallas_call(
        _dense_body,
        grid=grid,
        in_specs=in_specs,
        out_specs=pl.BlockSpec((_TILE, 128), edge),
        out_shape=jax.ShapeDtypeStruct((M, 128), jnp.float32),
    )(self_exp, gath, dn, *consts, w2p, pt, gamma, beta)


_NC, _NS = 2, 16
_NW = _NC * _NS
_CHUNK = 128


def _sc_gather(table, idx):
    M = idx.shape[0]
    per_w = M // _NW
    nch = per_w // _CHUNK
    idx3 = idx.reshape(_NW, nch, _CHUNK)
    mesh = plsc.VectorSubcoreMesh(core_axis_name="c", subcore_axis_name="s",
                                  num_cores=_NC, num_subcores=_NS)

    @functools.partial(
        pl.kernel,
        out_type=jax.ShapeDtypeStruct((M, 16), jnp.float32),
        mesh=mesh,
        compiler_params=pltpu.CompilerParams(use_tc_tiling_on_sc=False),
        scratch_types=[
            pltpu.VMEM((nch, _CHUNK), jnp.int32),
            pltpu.VMEM((per_w, 16), jnp.float32),
            pltpu.SemaphoreType.DMA,
        ],
    )
    def gather_k(table_hbm, idx_hbm, out_hbm, idx_v, rows_v, sem):
        wid = lax.axis_index("s") * _NC + lax.axis_index("c")
        pltpu.sync_copy(idx_hbm.at[wid], idx_v)
        for g in range(0, nch, 10):
            handles = [
                pltpu.async_copy(table_hbm.at[idx_v.at[c]],
                                 rows_v.at[pl.ds(c * _CHUNK, _CHUNK)], sem)
                for c in range(g, min(g + 10, nch))
            ]
            for h in handles:
                h.wait()
        pltpu.sync_copy(rows_v, out_hbm.at[pl.ds(wid * per_w, per_w)])

    return gather_k(table, idx3)


def _build_table(X, chain_idx):
    Nat = X[:, :, 0, :]
    Ca = X[:, :, 1, :]
    C = X[:, :, 2, :]
    O = X[:, :, 3, :]
    b = Ca - Nat
    c = C - Ca
    a = jnp.cross(b, c)
    Cb = -0.58273431 * a + 0.56802827 * b - 0.54067466 * c + Ca

    Bsz, L = chain_idx.shape
    change = jnp.concatenate(
        [jnp.zeros((Bsz, 1), dtype=jnp.int32),
         (chain_idx[:, 1:] != chain_idx[:, :-1]).astype(jnp.int32)], axis=1)
    rank = jnp.cumsum(change, axis=1)
    res = 100 * rank + jnp.arange(L, dtype=jnp.int32)[None, :]
    res = jnp.where(rank == rank[:, -1:], -100, res)
    combined = (4 * res + chain_idx.astype(jnp.int32)).astype(jnp.float32)

    return jnp.concatenate([Nat, Ca, C, O, Cb, combined[..., None]], axis=-1)


def kernel(X, chain_idx, mask, E_idx, D_neighbors,
           pos_W, pos_b, edge_W, ln_gamma, ln_beta):
    B, L, K = E_idx.shape
    M = B * L * K

    table = _build_table(X, chain_idx)
    table_flat = table.reshape(B * L, 16)
    idx_flat = (E_idx.astype(jnp.int32)
                + (jnp.arange(B, dtype=jnp.int32) * L)[:, None, None])
    gath = _sc_gather(table_flat, idx_flat.reshape(M))

    self_exp = jnp.broadcast_to(table[:, :, None, :], (B, L, K, 16))
    self_exp = self_exp.reshape(M, 16)
    dn = D_neighbors.reshape(M, 1)

    W1 = edge_W[:, :16]
    pt = (pos_W.T + pos_b[None, :]) @ W1.T
    pt = jnp.concatenate(
        [pt, jnp.zeros((6, 128), jnp.float32)], axis=0)
    pt = pt.astype(jnp.bfloat16)
    w2p = edge_W[:, 16:].T.astype(jnp.bfloat16)

    out = _dense_call(self_exp, gath, dn, w2p, pt,
                      ln_gamma.reshape(1, 128), ln_beta.reshape(1, 128))
    return out.reshape(B, L, K, 128)


# speedup vs baseline: 2.2538x; 1.4202x over previous
import functools

import numpy as np
import jax
import jax.numpy as jnp
from jax import lax
from jax.experimental import pallas as pl
from jax.experimental.pallas import tpu as pltpu
from jax.experimental.pallas import tpu_sc as plsc

NUM_RBF = 16
D_MIN, D_MAX = 2.0, 22.0
SIGMA = (D_MAX - D_MIN) / NUM_RBF

_PAIRS = [(0, 0), (2, 2), (3, 3), (4, 4), (1, 0), (1, 2), (1, 3), (1, 4),
          (0, 2), (0, 3), (0, 4), (4, 2), (4, 3), (3, 2), (0, 1), (2, 1),
          (3, 1), (4, 1), (2, 0), (3, 0), (4, 0), (2, 4), (3, 4), (2, 3)]

_BLOCK_OF_Q = np.zeros(25, np.int32)
for _p, (_a, _b) in enumerate(_PAIRS):
    _BLOCK_OF_Q[5 * _a + _b] = _p + 1
_BLOCK_OF_Q[6] = 0

_UA = np.zeros((16, 75), np.float32)
_UB = np.zeros((16, 75), np.float32)
_V3 = np.zeros((75, 25), np.float32)
for _q in range(25):
    _a, _b = _q // 5, _q % 5
    for _c in range(3):
        _UA[3 * _a + _c, 3 * _q + _c] = 1.0
        _UB[3 * _b + _c, 3 * _q + _c] = 1.0
        _V3[3 * _q + _c, _q] = 1.0

_MU = np.linspace(D_MIN, D_MAX, NUM_RBF).astype(np.float32)
_SEXP2 = np.zeros((25, 400), np.float32)
_MUS = np.zeros((1, 400), np.float32)
for _q in range(25):
    _blk = int(_BLOCK_OF_Q[_q])
    for _r in range(NUM_RBF):
        _SEXP2[_q, 16 * _blk + _r] = 1.0
for _blk in range(25):
    _MUS[0, 16 * _blk:16 * _blk + 16] = _MU
_INV_SIGMA = 1.0 / SIGMA


def _dense_body(self_ref, gath_ref, dn_ref, ua_ref, ub_ref, v3_ref,
                sexp_ref, mus_ref, w2_ref, pt_ref, gam_ref, bet_ref,
                out_ref):
    f32 = jnp.float32
    bf16 = jnp.bfloat16
    S = self_ref[...]
    N = gath_ref[...]
    dotb = functools.partial(jnp.dot, preferred_element_type=f32,
                             precision=lax.Precision.DEFAULT)

    def split_dot(x, m_ref):
        hi = x.astype(bf16)
        lo = (x - hi.astype(f32)).astype(bf16)
        return dotb(hi, m_ref[...]) + dotb(lo, m_ref[...])

    diff = split_dot(S, ua_ref) - split_dot(N, ub_ref)
    d2 = split_dot(diff * diff, v3_ref)
    dist = jnp.sqrt(d2 + 1e-06)

    T = dist.shape[0]
    lane25 = lax.broadcasted_iota(jnp.int32, (T, 25), 1)
    dn = jnp.broadcast_to(dn_ref[...], (T, 25))
    dfull = jnp.where(lane25 == 6, dn, dist)

    t = split_dot(dfull, sexp_ref)
    z = (t - mus_ref[...]) * _INV_SIGMA
    rbf = jnp.exp(-(z * z))
    e = dotb(rbf.astype(bf16), w2_ref[...])

    diffc = S[:, 15:16] - N[:, 15:16]
    offq = diffc * 0.25
    rem = diffc - 4.0 * jnp.floor(offq)
    d = jnp.where(rem == 0.0,
                  jnp.clip(offq + 32.0, 0.0, 64.0), 65.0)
    lane72 = lax.broadcasted_iota(jnp.int32, (T, 72), 1)
    onehot = (lane72 == d.astype(jnp.int32)).astype(bf16)
    e = e + dotb(onehot, pt_ref[...])

    mu = jnp.mean(e, axis=-1, keepdims=True)
    c = e - mu
    var = jnp.mean(c * c, axis=-1, keepdims=True)
    out_ref[...] = c * lax.rsqrt(var + 1e-05) * gam_ref[...] + bet_ref[...]


_TILE = 1024


def _dense_call(self_exp, gath, dn, w2p, pt, gamma, beta):
    M = self_exp.shape[0]
    grid = (M // _TILE,)
    consts = (jnp.asarray(_UA, dtype=jnp.bfloat16),
              jnp.asarray(_UB, dtype=jnp.bfloat16),
              jnp.asarray(_V3, dtype=jnp.bfloat16),
              jnp.asarray(_SEXP2, dtype=jnp.bfloat16), jnp.asarray(_MUS))
    edge = lambda i: (i, 0)
    full = lambda i: (0, 0)
    in_specs = [
        pl.BlockSpec((_TILE, 16), edge),
        pl.BlockSpec((_TILE, 16), edge),
        pl.BlockSpec((_TILE, 1), edge),
        pl.BlockSpec((16, 75), full),
        pl.BlockSpec((16, 75), full),
        pl.BlockSpec((75, 25), full),
        pl.BlockSpec((25, 400), full),
        pl.BlockSpec((1, 400), full),
        pl.BlockSpec((400, 128), full),
        pl.BlockSpec((72, 128), full),
        pl.BlockSpec((1, 128), full),
        pl.BlockSpec((1, 128), full),
    ]
    return pl.pallas_call(
        _dense_body,
        grid=grid,
        in_specs=in_specs,
        out_specs=pl.BlockSpec((_TILE, 128), edge),
        out_shape=jax.ShapeDtypeStruct((M, 128), jnp.float32),
    )(self_exp, gath, dn, *consts, w2p, pt, gamma, beta)


_NC, _NS = 2, 16
_NW = _NC * _NS
_CHUNK = 128


def _sc_gather(table, idx):
    M = idx.shape[0]
    per_w = M // _NW
    nch = per_w // _CHUNK
    idx3 = idx.reshape(_NW, nch, _CHUNK)
    mesh = plsc.VectorSubcoreMesh(core_axis_name="c", subcore_axis_name="s",
                                  num_cores=_NC, num_subcores=_NS)

    @functools.partial(
        pl.kernel,
        out_type=jax.ShapeDtypeStruct((M, 16), jnp.float32),
        mesh=mesh,
        compiler_params=pltpu.CompilerParams(use_tc_tiling_on_sc=False),
        scratch_types=[
            pltpu.VMEM((nch, _CHUNK), jnp.int32),
            pltpu.VMEM((per_w, 16), jnp.float32),
            pltpu.SemaphoreType.DMA,
        ],
    )
    def gather_k(table_hbm, idx_hbm, out_hbm, idx_v, rows_v, sem):
        wid = lax.axis_index("s") * _NC + lax.axis_index("c")
        pltpu.sync_copy(idx_hbm.at[wid], idx_v)
        for g in range(0, nch, 10):
            handles = [
                pltpu.async_copy(table_hbm.at[idx_v.at[c]],
                                 rows_v.at[pl.ds(c * _CHUNK, _CHUNK)], sem)
                for c in range(g, min(g + 10, nch))
            ]
            for h in handles:
                h.wait()
        pltpu.sync_copy(rows_v, out_hbm.at[pl.ds(wid * per_w, per_w)])

    return gather_k(table, idx3)


def _build_table(X, chain_idx):
    Nat = X[:, :, 0, :]
    Ca = X[:, :, 1, :]
    C = X[:, :, 2, :]
    O = X[:, :, 3, :]
    b = Ca - Nat
    c = C - Ca
    a = jnp.cross(b, c)
    Cb = -0.58273431 * a + 0.56802827 * b - 0.54067466 * c + Ca

    Bsz, L = chain_idx.shape
    change = jnp.concatenate(
        [jnp.zeros((Bsz, 1), dtype=jnp.int32),
         (chain_idx[:, 1:] != chain_idx[:, :-1]).astype(jnp.int32)], axis=1)
    rank = jnp.cumsum(change, axis=1)
    res = 100 * rank + jnp.arange(L, dtype=jnp.int32)[None, :]
    res = jnp.where(rank == rank[:, -1:], -100, res)
    combined = (4 * res + chain_idx.astype(jnp.int32)).astype(jnp.float32)

    return jnp.concatenate([Nat, Ca, C, O, Cb, combined[..., None]], axis=-1)


def kernel(X, chain_idx, mask, E_idx, D_neighbors,
           pos_W, pos_b, edge_W, ln_gamma, ln_beta):
    B, L, K = E_idx.shape
    M = B * L * K

    table = _build_table(X, chain_idx)
    table_flat = table.reshape(B * L, 16)
    idx_flat = (E_idx.astype(jnp.int32)
                + (jnp.arange(B, dtype=jnp.int32) * L)[:, None, None])
    gath = _sc_gather(table_flat, idx_flat.reshape(M))

    self_exp = jnp.broadcast_to(table[:, :, None, :], (B, L, K, 16))
    self_exp = self_exp.reshape(M, 16)
    dn = D_neighbors.reshape(M, 1)

    W1 = edge_W[:, :16]
    pt = (pos_W.T + pos_b[None, :]) @ W1.T
    pt = jnp.concatenate(
        [pt, jnp.zeros((6, 128), jnp.float32)], axis=0)
    pt = pt.astype(jnp.bfloat16)
    w2p = edge_W[:, 16:].T.astype(jnp.bfloat16)

    out = _dense_call(self_exp, gath, dn, w2p, pt,
                      ln_gamma.reshape(1, 128), ln_beta.reshape(1, 128))
    return out.reshape(B, L, K, 128)
